# baseline (speedup 1.0000x reference)
import functools

import jax
import jax.numpy as jnp
from jax import lax
from jax.experimental import pallas as pl
from jax.experimental.pallas import tpu as pltpu
from jax.experimental.pallas import tpu_sc as plsc

N = 10000
L = 16
NGROUPS = N // L
NW = 32
CHUNK_ROWS = 2000
NCHUNK = N // CHUNK_ROWS
EXTRA = NGROUPS - (NGROUPS // NW) * NW


_GATHER_DNUMS = lax.GatherDimensionNumbers(
    offset_dims=(), collapsed_slice_dims=(0,), start_index_map=(0,))


def _splat_lane(vec, lane):
    idx = jnp.full((L, 1), lane, dtype=jnp.int32)
    return lax.gather(vec, idx, _GATHER_DNUMS, (1,),
                      mode=lax.GatherScatterMode.PROMISE_IN_BOUNDS)


def _body(e_hbm, r_hbm, oval_hbm, oidx_hbm, e_v, buf_v, sval_v, sidx_v):
    c = lax.axis_index("c")
    s = lax.axis_index("s")
    w = s * 2 + c

    pltpu.sync_copy(e_hbm, e_v)

    def group_body(i, _):
        g = w + i * NW

        def chunk_body(k, carry):
            acc, idx = carry
            r0 = k * CHUNK_ROWS
            pltpu.sync_copy(r_hbm.at[pl.ds(r0, CHUNK_ROWS), g], buf_v)

            def sub_body(j, carry2):
                acc, idx = carry2
                base = r0 + j * L
                row_splat = jnp.full((L,), base, dtype=jnp.int32)
                e_vec = e_v[pl.ds(base, L)]
                for ii in range(L):
                    v = buf_v[j * L + ii]
                    es = _splat_lane(e_vec, ii)
                    p = v * es
                    m = p > acc
                    acc = jnp.where(m, p, acc)
                    idx = jnp.where(m, row_splat + ii, idx)
                return acc, idx

            return lax.fori_loop(0, CHUNK_ROWS // L, sub_body, (acc, idx))

        acc0 = jnp.zeros((L,), jnp.float32)
        idx0 = jnp.zeros((L,), jnp.int32)
        acc, idx = lax.fori_loop(0, NCHUNK, chunk_body, (acc0, idx0))
        sval_v[...] = acc
        sidx_v[...] = idx
        pltpu.sync_copy(sval_v, oval_hbm.at[g])
        pltpu.sync_copy(sidx_v, oidx_hbm.at[g])
        return 0

    ngroups_w = jnp.where(w < EXTRA, NGROUPS // NW + 1, NGROUPS // NW)
    lax.fori_loop(0, ngroups_w, group_body, 0)


@jax.jit
def _run(e, r3):
    mesh = plsc.VectorSubcoreMesh(core_axis_name="c", subcore_axis_name="s")
    f = functools.partial(
        pl.kernel,
        mesh=mesh,
        out_type=[
            jax.ShapeDtypeStruct((NGROUPS, L), jnp.float32),
            jax.ShapeDtypeStruct((NGROUPS, L), jnp.int32),
        ],
        scratch_types=[
            pltpu.VMEM((N,), jnp.float32),
            pltpu.VMEM((CHUNK_ROWS, L), jnp.float32),
            pltpu.VMEM((L,), jnp.float32),
            pltpu.VMEM((L,), jnp.int32),
        ],
        compiler_params=pltpu.CompilerParams(use_tc_tiling_on_sc=False),
    )(_body)
    return f(e, r3)


def kernel(embedding, r_embedding):
    e = embedding.reshape(N)
    r3 = r_embedding.reshape(N, NGROUPS, L)
    val, idx = _run(e, r3)
    return val.reshape(1, N), idx.reshape(N)


# speedup vs baseline: 3.9232x; 3.9232x over previous
import functools

import jax
import jax.numpy as jnp
from jax import lax
from jax.experimental import pallas as pl
from jax.experimental.pallas import tpu as pltpu
from jax.experimental.pallas import tpu_sc as plsc

N = 10000
L = 16
SW = 128
NSTRIPE = N // SW
NCOL_SC = NSTRIPE * SW
NW = 32
CHUNK_ROWS = 400
EXTRA = NSTRIPE - (NSTRIPE // NW) * NW

_GATHER_DNUMS = lax.GatherDimensionNumbers(
    offset_dims=(), collapsed_slice_dims=(0,), start_index_map=(0,))


def _splat_lane(vec, lane):
    idx = jnp.full((L, 1), lane, dtype=jnp.int32)
    return lax.gather(vec, idx, _GATHER_DNUMS, (1,),
                      mode=lax.GatherScatterMode.PROMISE_IN_BOUNDS)


def _body(e_hbm, r_hbm, oval_hbm, oidx_hbm, e_v, buf_v, sval_v, sidx_v):
    c = lax.axis_index("c")
    s = lax.axis_index("s")
    w = s * 2 + c

    pltpu.sync_copy(e_hbm, e_v)

    def stripe_body(i, _):
        stripe = w + i * NW
        c0 = stripe * SW

        def chunk_body(k, carry):
            r0 = k * CHUNK_ROWS
            pltpu.sync_copy(
                r_hbm.at[pl.ds(r0, CHUNK_ROWS), pl.ds(c0, SW)], buf_v)

            def grp(j, carry2):
                accs, idxs = carry2
                accs = list(accs)
                idxs = list(idxs)
                lbase = j * L
                gbase = r0 + lbase
                e_vec = e_v[pl.ds(gbase, L)]
                for ii in range(L):
                    es = _splat_lane(e_vec, ii)
                    rowv = jnp.full((L,), gbase + ii, dtype=jnp.int32)
                    for q in range(SW // L):
                        v = buf_v[lbase + ii, pl.ds(q * L, L)]
                        p = v * es
                        m = p > accs[q]
                        accs[q] = jnp.where(m, p, accs[q])
                        idxs[q] = jnp.where(m, rowv, idxs[q])
                return (tuple(accs), tuple(idxs))

            return lax.fori_loop(0, CHUNK_ROWS // L, grp, carry)

        z = tuple(jnp.zeros((L,), jnp.float32) for _ in range(SW // L))
        zi = tuple(jnp.zeros((L,), jnp.int32) for _ in range(SW // L))
        accs, idxs = lax.fori_loop(0, N // CHUNK_ROWS, chunk_body, (z, zi))
        for q in range(SW // L):
            sval_v[pl.ds(q * L, L)] = accs[q]
            sidx_v[pl.ds(q * L, L)] = idxs[q]
        pltpu.sync_copy(sval_v, oval_hbm.at[pl.ds(c0, SW)])
        pltpu.sync_copy(sidx_v, oidx_hbm.at[pl.ds(c0, SW)])
        return 0

    nstripes_w = jnp.where(w < EXTRA, NSTRIPE // NW + 1, NSTRIPE // NW)
    lax.fori_loop(0, nstripes_w, stripe_body, 0)


def _tail_body(e_ref, r_ref, val_ref, idx_ref):
    p = e_ref[...] * r_ref[...]
    m = jnp.max(p, axis=0, keepdims=True)
    rows = lax.broadcasted_iota(jnp.int32, p.shape, 0)
    cand = jnp.where(p == m, rows, N)
    val_ref[...] = m
    idx_ref[...] = jnp.min(cand, axis=0, keepdims=True)


@jax.jit
def _run(e, r):
    mesh = plsc.VectorSubcoreMesh(core_axis_name="c", subcore_axis_name="s")
    sc = functools.partial(
        pl.kernel,
        mesh=mesh,
        out_type=[
            jax.ShapeDtypeStruct((NCOL_SC,), jnp.float32),
            jax.ShapeDtypeStruct((NCOL_SC,), jnp.int32),
        ],
        scratch_types=[
            pltpu.VMEM((N,), jnp.float32),
            pltpu.VMEM((CHUNK_ROWS, SW), jnp.float32),
            pltpu.VMEM((SW,), jnp.float32),
            pltpu.VMEM((SW,), jnp.int32),
        ],
    )(_body)
    val_sc, idx_sc = sc(e, r)

    tc = pl.pallas_call(
        _tail_body,
        out_shape=[
            jax.ShapeDtypeStruct((1, N - NCOL_SC), jnp.float32),
            jax.ShapeDtypeStruct((1, N - NCOL_SC), jnp.int32),
        ],
    )
    val_tc, idx_tc = tc(e.reshape(N, 1), r[:, NCOL_SC:])

    val = jnp.concatenate([val_sc, val_tc.reshape(N - NCOL_SC)])
    idx = jnp.concatenate([idx_sc, idx_tc.reshape(N - NCOL_SC)])
    return val.reshape(1, N), idx


def kernel(embedding, r_embedding):
    val, idx = _run(embedding.reshape(N), r_embedding)
    return val, idx


# speedup vs baseline: 19.5215x; 4.9759x over previous
import functools

import jax
import jax.numpy as jnp
from jax import lax
from jax.experimental import pallas as pl
from jax.experimental.pallas import tpu as pltpu
from jax.experimental.pallas import tpu_sc as plsc

N = 10000
L = 16
SW = 128
NQ = SW // L
NPASS = 4
NSTRIPE = N // SW
NCOL_SC = NSTRIPE * SW
NW = 32
CHUNK_ROWS = 400
NCHUNK0 = 13
ROWS0 = NCHUNK0 * CHUNK_ROWS
NUNIT = 2 * NSTRIPE
EXTRA = NUNIT - (NUNIT // NW) * NW

_GATHER_DNUMS = lax.GatherDimensionNumbers(
    offset_dims=(), collapsed_slice_dims=(0,), start_index_map=(0,))


def _splat_lane(vec, lane):
    idx = jnp.full((L, 1), lane, dtype=jnp.int32)
    return lax.gather(vec, idx, _GATHER_DNUMS, (1,),
                      mode=lax.GatherScatterMode.PROMISE_IN_BOUNDS)


def _body(e_hbm, r_hbm, pval_hbm, pidx_hbm,
          e_v, buf0, buf1, sval_v, sidx_v, sem0, sem1):
    c = lax.axis_index("c")
    s = lax.axis_index("s")
    w = s * 2 + c

    pltpu.sync_copy(e_hbm, e_v)

    def unit_body(i, _):
        u = w + i * NW
        stripe = u // 2
        h = u - 2 * stripe
        c0 = stripe * SW
        r_base = h * ROWS0
        nch = NCHUNK0 - h

        zf = jnp.zeros((L,), jnp.float32)
        zi = jnp.zeros((L,), jnp.int32)
        for q in range(NQ):
            sval_v[pl.ds(q * L, L)] = zf
            sidx_v[pl.ds(q * L, L)] = zi

        def chunk_slice(k):
            return r_hbm.at[pl.ds(r_base + k * CHUNK_ROWS, CHUNK_ROWS),
                            pl.ds(c0, SW)]

        def start(k, buf, sem):
            pltpu.async_copy(chunk_slice(k), buf, sem)

        def wait(k, buf, sem):
            pltpu.make_async_copy(chunk_slice(k), buf, sem).wait()

        def process(k, buf):
            r0 = k * CHUNK_ROWS
            for p in range(NPASS):

                def jbody(j, carry):
                    accs, idxs = carry
                    accs = list(accs)
                    idxs = list(idxs)
                    lbase = j * L
                    gbase = r_base + r0 + lbase
                    e_vec = e_v[pl.ds(gbase, L)]
                    gsplat = jnp.full((L,), gbase, dtype=jnp.int32)
                    for ii in range(L):
                        es = _splat_lane(e_vec, ii)
                        rowv = gsplat + ii
                        for qq in range(NQ // NPASS):
                            q = (NQ // NPASS) * p + qq
                            v = buf[lbase + ii, pl.ds(q * L, L)]
                            pr = v * es
                            m = pr > accs[qq]
                            accs[qq] = jnp.where(m, pr, accs[qq])
                            idxs[qq] = jnp.where(m, rowv, idxs[qq])
                    return (tuple(accs), tuple(idxs))

                qs = [(NQ // NPASS) * p + qq for qq in range(NQ // NPASS)]
                acc0 = tuple(sval_v[pl.ds(q * L, L)] for q in qs)
                idx0 = tuple(sidx_v[pl.ds(q * L, L)] for q in qs)
                accs, idxs = lax.fori_loop(0, CHUNK_ROWS // L, jbody,
                                           (acc0, idx0))
                for qq, q in enumerate(qs):
                    sval_v[pl.ds(q * L, L)] = accs[qq]
                    sidx_v[pl.ds(q * L, L)] = idxs[qq]

        start(0, buf0, sem0)

        def kbody(k, _):
            even = (k % 2) == 0

            @pl.when(k + 1 < nch)
            def _():
                @pl.when(even)
                def _():
                    start(k + 1, buf1, sem1)

                @pl.when(jnp.logical_not(even))
                def _():
                    start(k + 1, buf0, sem0)

            @pl.when(even)
            def _():
                wait(k, buf0, sem0)
                process(k, buf0)

            @pl.when(jnp.logical_not(even))
            def _():
                wait(k, buf1, sem1)
                process(k, buf1)

            return 0

        lax.fori_loop(0, nch, kbody, 0)

        pltpu.sync_copy(sval_v, pval_hbm.at[pl.ds(u * SW, SW)])
        pltpu.sync_copy(sidx_v, pidx_hbm.at[pl.ds(u * SW, SW)])
        return 0

    nunits_w = jnp.where(w < EXTRA, NUNIT // NW + 1, NUNIT // NW)
    lax.fori_loop(0, nunits_w, unit_body, 0)


def _merge_body(pv_ref, pi_ref, e_ref, r_ref,
                mval_ref, midx_ref, tval_ref, tidx_ref):
    v0 = pv_ref[:, 0, :]
    v1 = pv_ref[:, 1, :]
    i0 = pi_ref[:, 0, :]
    i1 = pi_ref[:, 1, :]
    take1 = v1 > v0
    mval_ref[...] = jnp.where(take1, v1, v0)
    midx_ref[...] = jnp.where(take1, i1, i0)

    p = e_ref[...] * r_ref[...]
    m = jnp.max(p, axis=0, keepdims=True)
    rows = lax.broadcasted_iota(jnp.int32, p.shape, 0)
    cand = jnp.where(p == m, rows, N)
    tval_ref[...] = m
    tidx_ref[...] = jnp.min(cand, axis=0, keepdims=True)


@jax.jit
def _run(e, r):
    mesh = plsc.VectorSubcoreMesh(core_axis_name="c", subcore_axis_name="s")
    sc = functools.partial(
        pl.kernel,
        mesh=mesh,
        out_type=[
            jax.ShapeDtypeStruct((NUNIT * SW,), jnp.float32),
            jax.ShapeDtypeStruct((NUNIT * SW,), jnp.int32),
        ],
        scratch_types=[
            pltpu.VMEM((N,), jnp.float32),
            pltpu.VMEM((CHUNK_ROWS, SW), jnp.float32),
            pltpu.VMEM((CHUNK_ROWS, SW), jnp.float32),
            pltpu.VMEM((SW,), jnp.float32),
            pltpu.VMEM((SW,), jnp.int32),
            pltpu.SemaphoreType.DMA,
            pltpu.SemaphoreType.DMA,
        ],
    )(_body)
    pval, pidx = sc(e, r)

    tc = pl.pallas_call(
        _merge_body,
        out_shape=[
            jax.ShapeDtypeStruct((NSTRIPE, SW), jnp.float32),
            jax.ShapeDtypeStruct((NSTRIPE, SW), jnp.int32),
            jax.ShapeDtypeStruct((1, N - NCOL_SC), jnp.float32),
            jax.ShapeDtypeStruct((1, N - NCOL_SC), jnp.int32),
        ],
    )
    mval, midx, tval, tidx = tc(
        pval.reshape(NSTRIPE, 2, SW), pidx.reshape(NSTRIPE, 2, SW),
        e.reshape(N, 1), r[:, NCOL_SC:])

    val = jnp.concatenate([mval.reshape(NCOL_SC),
                           tval.reshape(N - NCOL_SC)])
    idx = jnp.concatenate([midx.reshape(NCOL_SC),
                           tidx.reshape(N - NCOL_SC)])
    return val.reshape(1, N), idx


def kernel(embedding, r_embedding):
    val, idx = _run(embedding.reshape(N), r_embedding)
    return val, idx
